# baseline (speedup 1.0000x reference)
import functools

import jax
import jax.numpy as jnp
from jax import lax
from jax.experimental import pallas as pl
from jax.experimental.pallas import tpu as pltpu
from jax.experimental.pallas import tpu_sc as plsc

N_TOKENS = 100000
D_MODEL = 768
MAX_LEN = 8192
BATCH = 4
SEQ = 4096

NC = 2
NS = 16
NW = NC * NS
LANES = 16

B_TOTAL = BATCH * SEQ
PER_W = B_TOTAL // NW
CHUNK = 64
N_CHUNKS = PER_W // CHUNK
D_VECS = D_MODEL // LANES


def _emb_kernel(ids_hbm, wemb_hbm, pos_hbm, out_hbm, idx_v, pos_v, row_v, sem):
    wid = lax.axis_index("s") * NC + lax.axis_index("c")
    base = wid * PER_W
    s_base = lax.rem(base, SEQ)

    def chunk_body(c, _):
        off = c * CHUNK
        pltpu.sync_copy(ids_hbm.at[pl.ds(base + off, CHUNK)], idx_v.at[c])
        pltpu.sync_copy(pos_hbm.at[pl.ds(s_base + off, CHUNK)], pos_v)
        pltpu.async_copy(wemb_hbm.at[idx_v.at[c]], row_v, sem).wait()

        def row_body(r, _):
            def vec_body(k, _):
                sl = pl.ds(k * LANES, LANES)
                row_v[r, sl] = row_v[r, sl] + pos_v[r, sl]
                return 0
            return lax.fori_loop(0, D_VECS, vec_body, 0)

        lax.fori_loop(0, CHUNK, row_body, 0)
        pltpu.sync_copy(row_v, out_hbm.at[pl.ds(base + off, CHUNK)])
        return 0

    lax.fori_loop(0, N_CHUNKS, chunk_body, 0)


def kernel(ids, word_emb, pos_table):
    ids_flat = ids.reshape(-1).astype(jnp.int32)
    mesh = plsc.VectorSubcoreMesh(core_axis_name="c", subcore_axis_name="s")
    out = pl.kernel(
        _emb_kernel,
        mesh=mesh,
        out_type=jax.ShapeDtypeStruct((B_TOTAL, D_MODEL), jnp.float32),
        scratch_types=[
            pltpu.VMEM((N_CHUNKS, CHUNK), jnp.int32),
            pltpu.VMEM((CHUNK, D_MODEL), jnp.float32),
            pltpu.VMEM((CHUNK, D_MODEL), jnp.float32),
            pltpu.SemaphoreType.DMA,
        ],
    )(ids_flat, word_emb, pos_table)
    return out.reshape(BATCH, SEQ, D_MODEL)


# speedup vs baseline: 2.3140x; 2.3140x over previous
import jax
import jax.numpy as jnp
from jax import lax
from jax.experimental import pallas as pl
from jax.experimental.pallas import tpu as pltpu
from jax.experimental.pallas import tpu_sc as plsc

N_TOKENS = 100000
D_MODEL = 768
MAX_LEN = 8192
BATCH = 4
SEQ = 4096

NC = 2
NS = 16
NW = NC * NS
LANES = 16

S_PER_W = SEQ // NW
CHUNK = 32
N_SBLK = S_PER_W // CHUNK
N_STEP = N_SBLK * BATCH
D_VECS = D_MODEL // LANES


def _emb_kernel(ids_hbm, wemb_hbm, pos_hbm, out_hbm,
                idx_v, pos_v, row_v,
                gs0, gs1, isem):
    wid = lax.axis_index("s") * NC + lax.axis_index("c")
    s0 = wid * S_PER_W
    gsem = (gs0, gs1)

    idx_copies = []
    for g in range(N_STEP):
        j, b = divmod(g, BATCH)
        idx_copies.append(pltpu.async_copy(
            ids_hbm.at[b, pl.ds(s0 + j * CHUNK, CHUNK)], idx_v.at[g], isem))
    for c in idx_copies:
        c.wait()

    gathers = {}

    def fire_gather(g):
        gathers[g] = pltpu.async_copy(
            wemb_hbm.at[idx_v.at[g]], row_v.at[g % 2], gsem[g % 2])

    def add_rows(p):
        def body(r, _):
            for k in range(D_VECS):
                sl = pl.ds(k * LANES, LANES)
                row_v[p, r, sl] = row_v[p, r, sl] + pos_v[r, sl]
            return 0
        lax.fori_loop(0, CHUNK, body, 0)

    fire_gather(0)
    for g in range(N_STEP):
        p = g % 2
        j, b = divmod(g, BATCH)
        if g + 1 < N_STEP:
            fire_gather(g + 1)
        if b == 0:
            pltpu.sync_copy(pos_hbm.at[pl.ds(s0 + j * CHUNK, CHUNK)], pos_v)
        gathers[g].wait()
        add_rows(p)
        pltpu.sync_copy(
            row_v.at[p],
            out_hbm.at[pl.ds(b * SEQ + s0 + j * CHUNK, CHUNK)])


def kernel(ids, word_emb, pos_table):
    ids32 = ids.astype(jnp.int32)
    mesh = plsc.VectorSubcoreMesh(core_axis_name="c", subcore_axis_name="s")
    out = pl.kernel(
        _emb_kernel,
        mesh=mesh,
        out_type=jax.ShapeDtypeStruct((BATCH * SEQ, D_MODEL), jnp.float32),
        scratch_types=[
            pltpu.VMEM((N_STEP, CHUNK), jnp.int32),
            pltpu.VMEM((CHUNK, D_MODEL), jnp.float32),
            pltpu.VMEM((2, CHUNK, D_MODEL), jnp.float32),
        ] + [pltpu.SemaphoreType.DMA] * 3,
    )(ids32, word_emb, pos_table)
    return out.reshape(BATCH, SEQ, D_MODEL)
